# initial kernel scaffold (unmeasured)
import jax
import jax.numpy as jnp
from jax import lax
from jax.experimental import pallas as pl
from jax.experimental.pallas import tpu as pltpu

N_DEV = 4
HGRP = 4


def kernel(Q, K, V):
    B, S, H, D = Q.shape
    scale = D ** -0.5

    def body(q_ref, k_ref, v_ref, out_ref, kg_ref, vg_ref,
             ksend, krecv, vsend, vrecv):
        my = lax.axis_index("i")
        right = lax.rem(my + 1, N_DEV)
        left = lax.rem(my + N_DEV - 1, N_DEV)

        barrier = pltpu.get_barrier_semaphore()
        for nbr in (left, right):
            pl.semaphore_signal(barrier, inc=1, device_id=(nbr,),
                                device_id_type=pl.DeviceIdType.MESH)
        pl.semaphore_wait(barrier, 2)

        kg_ref[pl.ds(my, 1)] = k_ref[...].reshape(1, B, S, H, D)
        vg_ref[pl.ds(my, 1)] = v_ref[...].reshape(1, B, S, H, D)

        for h in range(N_DEV - 1):
            src_o = lax.rem(my - h + N_DEV, N_DEV)
            kr = pltpu.make_async_remote_copy(
                src_ref=kg_ref.at[pl.ds(src_o, 1)],
                dst_ref=kg_ref.at[pl.ds(src_o, 1)],
                send_sem=ksend.at[h], recv_sem=krecv.at[h],
                device_id=(right,), device_id_type=pl.DeviceIdType.MESH)
            vr = pltpu.make_async_remote_copy(
                src_ref=vg_ref.at[pl.ds(src_o, 1)],
                dst_ref=vg_ref.at[pl.ds(src_o, 1)],
                send_sem=vsend.at[h], recv_sem=vrecv.at[h],
                device_id=(right,), device_id_type=pl.DeviceIdType.MESH)
            kr.start()
            vr.start()
            kr.wait()
            vr.wait()

        for b in range(B):
            for g in range(H // HGRP):
                hs = g * HGRP
                q = q_ref[b, :, hs:hs + HGRP, :]
                s_parts = []
                for o in range(N_DEV):
                    k_o = kg_ref[o, b, :, hs:hs + HGRP, :]
                    s_parts.append(jnp.einsum(
                        "qhd,khd->hqk", q, k_o,
                        preferred_element_type=jnp.float32))
                s = jnp.concatenate(s_parts, axis=-1) * scale
                m = jnp.max(s, axis=-1, keepdims=True)
                p = jnp.exp(s - m)
                p = p / jnp.sum(p, axis=-1, keepdims=True)
                acc = jnp.zeros((S, HGRP, D), jnp.float32)
                for o in range(N_DEV):
                    v_o = vg_ref[o, b, :, hs:hs + HGRP, :]
                    acc = acc + jnp.einsum(
                        "hqk,khd->qhd", p[:, :, o * S:(o + 1) * S], v_o,
                        preferred_element_type=jnp.float32)
                out_ref[b, :, hs:hs + HGRP, :] = acc

    return pl.pallas_call(
        body,
        out_shape=jax.ShapeDtypeStruct((B, S, H, D), jnp.float32),
        in_specs=[pl.BlockSpec(memory_space=pltpu.VMEM)] * 3,
        out_specs=pl.BlockSpec(memory_space=pltpu.VMEM),
        scratch_shapes=[
            pltpu.VMEM((N_DEV, B, S, H, D), jnp.float32),
            pltpu.VMEM((N_DEV, B, S, H, D), jnp.float32),
            pltpu.SemaphoreType.DMA((N_DEV - 1,)),
            pltpu.SemaphoreType.DMA((N_DEV - 1,)),
            pltpu.SemaphoreType.DMA((N_DEV - 1,)),
            pltpu.SemaphoreType.DMA((N_DEV - 1,)),
        ],
        compiler_params=pltpu.CompilerParams(collective_id=0),
    )(Q, K, V)


# baseline (device time: 653403 ns/iter reference)
import jax
import jax.numpy as jnp
from jax import lax
from jax.experimental import pallas as pl
from jax.experimental.pallas import tpu as pltpu

N_DEV = 4


def _gather_kv(K, V):
    B, S, H, D = K.shape

    def body(k_ref, v_ref, kg_ref, vg_ref, fillsem, ksend, krecv, vsend, vrecv):
        my = lax.axis_index("i")
        right = lax.rem(my + 1, N_DEV)
        left = lax.rem(my + N_DEV - 1, N_DEV)

        kfill = pltpu.make_async_copy(k_ref, kg_ref.at[my], fillsem.at[0])
        vfill = pltpu.make_async_copy(v_ref, vg_ref.at[my], fillsem.at[1])
        kfill.start()
        vfill.start()
        kfill.wait()
        vfill.wait()

        barrier = pltpu.get_barrier_semaphore()
        for nbr in (left, right):
            pl.semaphore_signal(barrier, inc=1, device_id=(nbr,),
                                device_id_type=pl.DeviceIdType.MESH)
        pl.semaphore_wait(barrier, 2)

        for h in range(N_DEV - 1):
            src_o = lax.rem(my - h + N_DEV, N_DEV)
            kr = pltpu.make_async_remote_copy(
                src_ref=kg_ref.at[pl.ds(src_o, 1)],
                dst_ref=kg_ref.at[pl.ds(src_o, 1)],
                send_sem=ksend.at[h], recv_sem=krecv.at[h],
                device_id=(right,), device_id_type=pl.DeviceIdType.MESH)
            vr = pltpu.make_async_remote_copy(
                src_ref=vg_ref.at[pl.ds(src_o, 1)],
                dst_ref=vg_ref.at[pl.ds(src_o, 1)],
                send_sem=vsend.at[h], recv_sem=vrecv.at[h],
                device_id=(right,), device_id_type=pl.DeviceIdType.MESH)
            kr.start()
            vr.start()
            kr.wait()
            vr.wait()

    return pl.pallas_call(
        body,
        out_shape=[
            jax.ShapeDtypeStruct((N_DEV, B, S, H, D), jnp.float32),
            jax.ShapeDtypeStruct((N_DEV, B, S, H, D), jnp.float32),
        ],
        in_specs=[pl.BlockSpec(memory_space=pl.ANY)] * 2,
        out_specs=[pl.BlockSpec(memory_space=pl.ANY)] * 2,
        scratch_shapes=[
            pltpu.SemaphoreType.DMA((2,)),
            pltpu.SemaphoreType.DMA((N_DEV - 1,)),
            pltpu.SemaphoreType.DMA((N_DEV - 1,)),
            pltpu.SemaphoreType.DMA((N_DEV - 1,)),
            pltpu.SemaphoreType.DMA((N_DEV - 1,)),
        ],
        compiler_params=pltpu.CompilerParams(collective_id=0),
    )(K, V)


def _attention(Q, KG, VG):
    B, S, H, D = Q.shape
    scale = D ** -0.5

    def body(q_ref, k_ref, v_ref, o_ref):
        for h in range(H):
            q_h = q_ref[0, :, h, :]
            ks = jnp.concatenate(
                [k_ref[o, 0, :, h, :] for o in range(N_DEV)], axis=0)
            vs = jnp.concatenate(
                [v_ref[o, 0, :, h, :] for o in range(N_DEV)], axis=0)
            s = lax.dot_general(
                q_h, ks, (((1,), (1,)), ((), ())),
                preferred_element_type=jnp.float32) * scale
            m = jnp.max(s, axis=-1, keepdims=True)
            p = jnp.exp(s - m)
            p = p / jnp.sum(p, axis=-1, keepdims=True)
            o_ref[0, :, h, :] = lax.dot_general(
                p, vs, (((1,), (0,)), ((), ())),
                preferred_element_type=jnp.float32)

    return pl.pallas_call(
        body,
        grid=(B,),
        out_shape=jax.ShapeDtypeStruct((B, S, H, D), jnp.float32),
        in_specs=[
            pl.BlockSpec((1, S, H, D), lambda b: (b, 0, 0, 0)),
            pl.BlockSpec((N_DEV, 1, S, H, D), lambda b: (0, b, 0, 0, 0)),
            pl.BlockSpec((N_DEV, 1, S, H, D), lambda b: (0, b, 0, 0, 0)),
        ],
        out_specs=pl.BlockSpec((1, S, H, D), lambda b: (b, 0, 0, 0)),
        compiler_params=pltpu.CompilerParams(
            vmem_limit_bytes=60 * 1024 * 1024),
    )(Q, KG, VG)


def kernel(Q, K, V):
    KG, VG = _gather_kv(K, V)
    return _attention(Q, KG, VG)
